# baseline (speedup 1.0000x reference)
import functools

import jax
import jax.numpy as jnp
from jax import lax
from jax.experimental import pallas as pl
from jax.experimental.pallas import tpu as pltpu
from jax.experimental.pallas import tpu_sc as plsc

N = 100000
NUM_WORKERS = 32
PER_W = 3128
LAST_BASE = N - PER_W
C = 128
NFULL = PER_W // C
TAIL = PER_W - NFULL * C
K = 3
NSET = 2 * K
NPOS = NFULL // NSET
DIMS = (16, 32, 32, 48)
COLS = (0, 16, 48, 80)
DOUT = 128

_mesh = plsc.VectorSubcoreMesh(core_axis_name="c", subcore_axis_name="s")

_scratch = [pltpu.VMEM((4, PER_W), jnp.int32)]
for _k in range(NSET):
    _scratch.extend(pltpu.VMEM((C, d), jnp.float32) for d in DIMS)
_scratch.extend(pltpu.VMEM((TAIL, d), jnp.float32) for d in DIMS)
_scratch.extend(pltpu.SemaphoreType.DMA for _ in range(2 * NSET + 1))


@functools.partial(
    pl.kernel,
    out_type=jax.ShapeDtypeStruct((N, DOUT), jnp.float32),
    mesh=_mesh,
    scratch_types=_scratch,
    compiler_params=pltpu.CompilerParams(use_tc_tiling_on_sc=False),
)
def _sc_lookup(idx0, idx1, idx2, idx3, t0, t1, t2, t3, out, *s):
    idxs = (idx0, idx1, idx2, idx3)
    tables = (t0, t1, t2, t3)
    iv = s[0]
    bufs = [list(s[1 + 4 * k:5 + 4 * k]) for k in range(NSET)]
    trvs = list(s[1 + 4 * NSET:5 + 4 * NSET])
    gsems = s[5 + 4 * NSET:5 + 5 * NSET]
    wsems = s[5 + 5 * NSET:5 + 6 * NSET]
    tsem = s[5 + 6 * NSET]

    wid = lax.axis_index("s") * 2 + lax.axis_index("c")
    base = jnp.minimum(wid * PER_W, LAST_BASE)

    for l in range(4):
        pltpu.sync_copy(idxs[l].at[pl.ds(base, PER_W)], iv.at[l])

    def fire(off, n, rvs, sem):
        for l in range(4):
            pltpu.async_copy(tables[l].at[iv.at[l, pl.ds(off, n)]], rvs[l], sem)

    def drain_gather(off, n, rvs, sem):
        for l in range(4):
            pltpu.make_async_copy(
                tables[l].at[iv.at[l, pl.ds(off, n)]], rvs[l], sem
            ).wait()

    def out_band(off, n, l):
        return out.at[pl.ds(base + off, n), pl.ds(COLS[l], DIMS[l])]

    def fire_writes(off, n, rvs, sem):
        for l in range(4):
            pltpu.async_copy(rvs[l], out_band(off, n, l), sem)

    def drain_writes(off, n, rvs, sem):
        for l in range(4):
            pltpu.make_async_copy(rvs[l], out_band(off, n, l), sem).wait()

    for m in range(K):
        fire(m * C, C, bufs[m], gsems[m])
    fire(NFULL * C, TAIL, trvs, tsem)

    @pl.loop(0, NPOS)
    def _(p):
        for m in range(NSET):
            jj = p * NSET + m
            off = jj * C
            drain_gather(off, C, bufs[m], gsems[m])
            fire_writes(off, C, bufs[m], wsems[m])
            m2 = (m + K) % NSET

            @pl.when(jj + K < NFULL)
            def _():
                @pl.when(jj >= K)
                def _():
                    drain_writes((jj - K) * C, C, bufs[m2], wsems[m2])

                fire((jj + K) * C, C, bufs[m2], gsems[m2])

    for jj in range(NFULL - NSET, NFULL):
        m = jj % NSET
        drain_writes(jj * C, C, bufs[m], wsems[m])

    drain_gather(NFULL * C, TAIL, trvs, tsem)
    for l in range(4):
        pltpu.sync_copy(trvs[l], out_band(NFULL * C, TAIL, l))


def kernel(code_levels, table_0, table_1, table_2, table_3):
    return _sc_lookup(
        code_levels[:, 0], code_levels[:, 1],
        code_levels[:, 2], code_levels[:, 3],
        table_0, table_1, table_2, table_3,
    )
